# baseline (device time: 126492 ns/iter reference)
import jax
import jax.numpy as jnp
from jax import lax
from jax.experimental import pallas as pl
from jax.experimental.pallas import tpu as pltpu

N_DEV = 16
SQ = 128
SKV = 128
DH = 64


def kernel(x, Wq, K_ext, V_ext, Wo):
    B, Sq, Dm = x.shape
    H_loc = Wq.shape[1] // DH
    H = N_DEV * H_loc
    BH = B * H_loc

    my = lax.axis_index("i")

    k_loc = lax.dynamic_slice_in_dim(K_ext, my * B, B, axis=0)
    v_loc = lax.dynamic_slice_in_dim(V_ext, my * B, B, axis=0)
    kp = k_loc.transpose(2, 0, 1, 3).reshape(H * B, SKV, DH).astype(jnp.bfloat16)
    vp = v_loc.transpose(2, 0, 1, 3).reshape(H * B, SKV, DH).astype(jnp.bfloat16)

    x2d = x.reshape(B * Sq, Dm).astype(jnp.bfloat16)
    wq_b = Wq.astype(jnp.bfloat16)
    wo_b = Wo.astype(jnp.bfloat16)

    def body(x_ref, wq_ref, k_ref, v_ref, wo_ref, o_ref,
             wqb, wob, q_send, q_recv, o_send, o_recv):
        me = lax.axis_index("i")
        left = lax.rem(me + N_DEV - 1, N_DEV)
        right = lax.rem(me + 1, N_DEV)

        barrier_sem = pltpu.get_barrier_semaphore()
        for nbr in (left, right):
            pl.semaphore_signal(
                barrier_sem, inc=1,
                device_id=(nbr,), device_id_type=pl.DeviceIdType.MESH,
            )
        pl.semaphore_wait(barrier_sem, 2)

        wqb[me] = wq_ref[...]
        wob[me] = wo_ref[...]

        xv = x_ref[...]

        qb = lax.broadcasted_iota(jnp.int32, (SQ, SKV), 0) // 64
        kb = lax.broadcasted_iota(jnp.int32, (SQ, SKV), 1) // 64
        mask = (qb == kb) | (kb == 0) | ((qb + kb) % 3 == 0)

        def block_contrib(blk):
            wq_blk = wqb[blk]
            q = jnp.dot(xv, wq_blk, preferred_element_type=jnp.float32)
            q = q.astype(jnp.bfloat16)
            q4 = q.reshape(B, Sq, H_loc, DH).transpose(2, 0, 1, 3)
            q4 = q4.reshape(BH, Sq, DH)
            k_blk = k_ref[pl.ds(blk * BH, BH)]
            v_blk = v_ref[pl.ds(blk * BH, BH)]
            scores = lax.dot_general(
                q4, k_blk, (((2,), (2,)), ((0,), (0,))),
                preferred_element_type=jnp.float32,
            ) * 0.125
            scores = jnp.where(mask[None], scores, -1e9)
            m = jnp.max(scores, axis=-1, keepdims=True)
            e = jnp.exp(scores - m)
            w = (e / jnp.sum(e, axis=-1, keepdims=True)).astype(jnp.bfloat16)
            ctx = lax.dot_general(
                w, v_blk, (((2,), (1,)), ((0,), (0,))),
                preferred_element_type=jnp.float32,
            )
            ctx = ctx.astype(jnp.bfloat16).reshape(H_loc, B, Sq, DH)
            ctx2d = ctx.transpose(1, 2, 0, 3).reshape(B * Sq, H_loc * DH)
            wo_blk = wob[blk]
            return jnp.dot(ctx2d, wo_blk, preferred_element_type=jnp.float32)

        acc = jnp.zeros((B * Sq, Dm), jnp.float32)
        for h in range(N_DEV):
            blk = lax.rem(me - h + N_DEV, N_DEV)
            if h < N_DEV - 1:
                rq = pltpu.make_async_remote_copy(
                    src_ref=wqb.at[blk], dst_ref=wqb.at[blk],
                    send_sem=q_send.at[h], recv_sem=q_recv.at[h],
                    device_id=(right,), device_id_type=pl.DeviceIdType.MESH,
                )
                ro = pltpu.make_async_remote_copy(
                    src_ref=wob.at[blk], dst_ref=wob.at[blk],
                    send_sem=o_send.at[h], recv_sem=o_recv.at[h],
                    device_id=(right,), device_id_type=pl.DeviceIdType.MESH,
                )
                rq.start()
                ro.start()
            acc = acc + block_contrib(blk)
            if h < N_DEV - 1:
                rq.wait()
                ro.wait()
        o_ref[...] = acc

    out2d = pl.pallas_call(
        body,
        out_shape=jax.ShapeDtypeStruct((B * Sq, Dm), jnp.float32),
        in_specs=[pl.BlockSpec(memory_space=pltpu.VMEM)] * 5,
        out_specs=pl.BlockSpec(memory_space=pltpu.VMEM),
        scratch_shapes=[
            pltpu.VMEM((N_DEV, Dm, H_loc * DH), jnp.bfloat16),
            pltpu.VMEM((N_DEV, H_loc * DH, Dm), jnp.bfloat16),
            pltpu.SemaphoreType.DMA((N_DEV - 1,)),
            pltpu.SemaphoreType.DMA((N_DEV - 1,)),
            pltpu.SemaphoreType.DMA((N_DEV - 1,)),
            pltpu.SemaphoreType.DMA((N_DEV - 1,)),
        ],
        compiler_params=pltpu.CompilerParams(collective_id=0),
    )(x2d, wq_b, kp, vp, wo_b)

    return out2d.reshape(B, Sq, Dm)


# device time: 86677 ns/iter; 1.4593x vs baseline; 1.4593x over previous
import jax
import jax.numpy as jnp
from jax import lax
from jax.experimental import pallas as pl
from jax.experimental.pallas import tpu as pltpu

N_DEV = 16
RH = N_DEV // 2
LH = N_DEV // 2 - 1
SQ = 128
SKV = 128
DH = 64


def kernel(x, Wq, K_ext, V_ext, Wo):
    B, Sq, Dm = x.shape
    H_loc = Wq.shape[1] // DH
    H = N_DEV * H_loc
    BH = B * H_loc

    my = lax.axis_index("i")

    k_loc = lax.dynamic_slice_in_dim(K_ext, my * B, B, axis=0)
    v_loc = lax.dynamic_slice_in_dim(V_ext, my * B, B, axis=0)
    kp = k_loc.transpose(2, 0, 1, 3).reshape(H * B, SKV, DH).astype(jnp.bfloat16)
    vp = v_loc.transpose(2, 0, 1, 3).reshape(H * B, SKV, DH).astype(jnp.bfloat16)

    x2d = x.reshape(B * Sq, Dm).astype(jnp.bfloat16)
    wq_b = Wq.astype(jnp.bfloat16)
    wo_b = Wo.astype(jnp.bfloat16)

    def body(x_ref, wq_ref, k_ref, v_ref, wo_ref, o_ref,
             wqb, wob, qr_send, qr_recv, or_send, or_recv,
             ql_send, ql_recv, ol_send, ol_recv):
        me = lax.axis_index("i")
        left = lax.rem(me + N_DEV - 1, N_DEV)
        right = lax.rem(me + 1, N_DEV)

        barrier_sem = pltpu.get_barrier_semaphore()
        for nbr in (left, right):
            pl.semaphore_signal(
                barrier_sem, inc=1,
                device_id=(nbr,), device_id_type=pl.DeviceIdType.MESH,
            )
        pl.semaphore_wait(barrier_sem, 2)

        wqb[me] = wq_ref[...]
        wob[me] = wo_ref[...]

        xv = x_ref[...]

        qb = lax.broadcasted_iota(jnp.int32, (SQ, SKV), 0) // 64
        kb = lax.broadcasted_iota(jnp.int32, (SQ, SKV), 1) // 64
        mask = (qb == kb) | (kb == 0) | ((qb + kb) % 3 == 0)

        def block_contrib(blk):
            wq_blk = wqb[blk]
            q = jnp.dot(xv, wq_blk, preferred_element_type=jnp.float32)
            q = q.astype(jnp.bfloat16)
            q4 = q.reshape(B, Sq, H_loc, DH).transpose(2, 0, 1, 3)
            q4 = q4.reshape(BH, Sq, DH)
            k_blk = k_ref[pl.ds(blk * BH, BH)]
            v_blk = v_ref[pl.ds(blk * BH, BH)]
            scores = lax.dot_general(
                q4, k_blk, (((2,), (2,)), ((0,), (0,))),
                preferred_element_type=jnp.float32,
            ) * 0.125
            scores = jnp.where(mask[None], scores, -1e9)
            m = jnp.max(scores, axis=-1, keepdims=True)
            e = jnp.exp(scores - m)
            w = (e / jnp.sum(e, axis=-1, keepdims=True)).astype(jnp.bfloat16)
            ctx = lax.dot_general(
                w, v_blk, (((2,), (1,)), ((0,), (0,))),
                preferred_element_type=jnp.float32,
            )
            ctx = ctx.astype(jnp.bfloat16).reshape(H_loc, B, Sq, DH)
            ctx2d = ctx.transpose(1, 2, 0, 3).reshape(B * Sq, H_loc * DH)
            wo_blk = wob[blk]
            return jnp.dot(ctx2d, wo_blk, preferred_element_type=jnp.float32)

        acc = jnp.zeros((B * Sq, Dm), jnp.float32)
        for t in range(RH):
            blk_r = lax.rem(me - t + N_DEV, N_DEV)
            blk_l = lax.rem(me + t, N_DEV)
            rdmas = []
            for buf, ss, rs in ((wqb, qr_send, qr_recv), (wob, or_send, or_recv)):
                r = pltpu.make_async_remote_copy(
                    src_ref=buf.at[blk_r], dst_ref=buf.at[blk_r],
                    send_sem=ss.at[t], recv_sem=rs.at[t],
                    device_id=(right,), device_id_type=pl.DeviceIdType.MESH,
                )
                r.start()
                rdmas.append(r)
            if t < LH:
                for buf, ss, rs in ((wqb, ql_send, ql_recv), (wob, ol_send, ol_recv)):
                    r = pltpu.make_async_remote_copy(
                        src_ref=buf.at[blk_l], dst_ref=buf.at[blk_l],
                        send_sem=ss.at[t], recv_sem=rs.at[t],
                        device_id=(left,), device_id_type=pl.DeviceIdType.MESH,
                    )
                    r.start()
                    rdmas.append(r)
            if t == 0:
                acc = acc + block_contrib(me)
            else:
                acc = acc + block_contrib(blk_r)
                acc = acc + block_contrib(blk_l)
            for r in rdmas:
                r.wait()
        acc = acc + block_contrib(lax.rem(me + RH, N_DEV))
        o_ref[...] = acc

    out2d = pl.pallas_call(
        body,
        out_shape=jax.ShapeDtypeStruct((B * Sq, Dm), jnp.float32),
        in_specs=[pl.BlockSpec(memory_space=pltpu.VMEM)] * 5,
        out_specs=pl.BlockSpec(memory_space=pltpu.VMEM),
        scratch_shapes=[
            pltpu.VMEM((N_DEV, Dm, H_loc * DH), jnp.bfloat16),
            pltpu.VMEM((N_DEV, H_loc * DH, Dm), jnp.bfloat16),
            pltpu.SemaphoreType.DMA((RH,)),
            pltpu.SemaphoreType.DMA((RH,)),
            pltpu.SemaphoreType.DMA((RH,)),
            pltpu.SemaphoreType.DMA((RH,)),
            pltpu.SemaphoreType.DMA((LH,)),
            pltpu.SemaphoreType.DMA((LH,)),
            pltpu.SemaphoreType.DMA((LH,)),
            pltpu.SemaphoreType.DMA((LH,)),
        ],
        compiler_params=pltpu.CompilerParams(collective_id=0),
    )(x2d, wq_b, kp, vp, wo_b)

    return out2d.reshape(B, Sq, Dm)


# device time: 75440 ns/iter; 1.6767x vs baseline; 1.1490x over previous
import jax
import jax.numpy as jnp
from jax import lax
from jax.experimental import pallas as pl
from jax.experimental.pallas import tpu as pltpu

N_DEV = 16
RH = N_DEV // 2
LH = N_DEV // 2 - 1
SQ = 128
SKV = 128
DH = 64


def kernel(x, Wq, K_ext, V_ext, Wo):
    B, Sq, Dm = x.shape
    H_loc = Wq.shape[1] // DH
    H = N_DEV * H_loc
    BH = B * H_loc

    my = lax.axis_index("i")

    k_loc = lax.dynamic_slice_in_dim(K_ext, my * B, B, axis=0)
    v_loc = lax.dynamic_slice_in_dim(V_ext, my * B, B, axis=0)
    kp = k_loc.transpose(2, 0, 1, 3).reshape(H * B, SKV, DH).astype(jnp.bfloat16)
    vp = v_loc.transpose(2, 0, 1, 3).reshape(H * B, SKV, DH).astype(jnp.bfloat16)

    x2d = x.reshape(B * Sq, Dm).astype(jnp.bfloat16)
    wq_b = Wq.astype(jnp.bfloat16)
    wo_b = Wo.astype(jnp.bfloat16)

    def body(x_ref, wq_ref, k_ref, v_ref, wo_ref, o_ref,
             wqb, wob, qr_send, qr_recv, or_send, or_recv,
             ql_send, ql_recv, ol_send, ol_recv):
        me = lax.axis_index("i")
        left = lax.rem(me + N_DEV - 1, N_DEV)
        right = lax.rem(me + 1, N_DEV)

        barrier_sem = pltpu.get_barrier_semaphore()
        for nbr in (left, right):
            pl.semaphore_signal(
                barrier_sem, inc=1,
                device_id=(nbr,), device_id_type=pl.DeviceIdType.MESH,
            )
        pl.semaphore_wait(barrier_sem, 2)

        wqb[me] = wq_ref[...]
        wob[me] = wo_ref[...]

        xv = x_ref[...]

        qb = lax.broadcasted_iota(jnp.int32, (SQ, SKV), 0) // 64
        kb = lax.broadcasted_iota(jnp.int32, (SQ, SKV), 1) // 64
        mask = (qb == kb) | (kb == 0) | ((qb + kb) % 3 == 0)

        def block_contrib(blk):
            wq_blk = wqb[blk]
            q = jnp.dot(xv, wq_blk, preferred_element_type=jnp.float32)
            q = q.astype(jnp.bfloat16)
            q4 = q.reshape(B, Sq, H_loc, DH).transpose(2, 0, 1, 3)
            q4 = q4.reshape(BH, Sq, DH)
            k_blk = k_ref[pl.ds(blk * BH, BH)]
            v_blk = v_ref[pl.ds(blk * BH, BH)]
            scores = lax.dot_general(
                q4, k_blk, (((2,), (2,)), ((0,), (0,))),
                preferred_element_type=jnp.float32,
            ) * 0.125
            scores = jnp.where(mask[None], scores, -1e9)
            m = jnp.max(scores, axis=-1, keepdims=True)
            e = jnp.exp(scores - m)
            w = (e / jnp.sum(e, axis=-1, keepdims=True)).astype(jnp.bfloat16)
            ctx = lax.dot_general(
                w, v_blk, (((2,), (1,)), ((0,), (0,))),
                preferred_element_type=jnp.float32,
            )
            ctx = ctx.astype(jnp.bfloat16).reshape(H_loc, B, Sq, DH)
            ctx2d = ctx.transpose(1, 2, 0, 3).reshape(B * Sq, H_loc * DH)
            wo_blk = wob[blk]
            return jnp.dot(ctx2d, wo_blk, preferred_element_type=jnp.float32)

        import os as _os
        if _os.environ.get("KERNEL_NO_RDMA"):
            acc = jnp.zeros((B * Sq, Dm), jnp.float32)
            for h in range(N_DEV):
                acc = acc + block_contrib(lax.rem(me + h, N_DEV))
            o_ref[...] = acc
            return

        def send(buf, blk, ss, rs, t, dev):
            r = pltpu.make_async_remote_copy(
                src_ref=buf.at[blk], dst_ref=buf.at[blk],
                send_sem=ss.at[t], recv_sem=rs.at[t],
                device_id=(dev,), device_id_type=pl.DeviceIdType.MESH,
            )
            r.start()
            return r

        pending = []
        pending.append(send(wqb, me, qr_send, qr_recv, 0, right))
        pending.append(send(wob, me, or_send, or_recv, 0, right))
        pending.append(send(wqb, me, ql_send, ql_recv, 0, left))
        pending.append(send(wob, me, ol_send, ol_recv, 0, left))
        acc = block_contrib(me)

        for t in range(1, RH + 1):
            blk_r = lax.rem(me - t + N_DEV, N_DEV)
            blk_l = lax.rem(me + t, N_DEV)
            for buf, ss, rs in ((wqb, qr_send, qr_recv), (wob, or_send, or_recv)):
                pltpu.make_async_remote_copy(
                    src_ref=buf.at[blk_r], dst_ref=buf.at[blk_r],
                    send_sem=ss.at[t - 1], recv_sem=rs.at[t - 1],
                    device_id=(left,), device_id_type=pl.DeviceIdType.MESH,
                ).wait_recv()
            if t < RH:
                pending.append(send(wqb, blk_r, qr_send, qr_recv, t, right))
                pending.append(send(wob, blk_r, or_send, or_recv, t, right))
            if t <= LH:
                for buf, ss, rs in ((wqb, ql_send, ql_recv), (wob, ol_send, ol_recv)):
                    pltpu.make_async_remote_copy(
                        src_ref=buf.at[blk_l], dst_ref=buf.at[blk_l],
                        send_sem=ss.at[t - 1], recv_sem=rs.at[t - 1],
                        device_id=(right,), device_id_type=pl.DeviceIdType.MESH,
                    ).wait_recv()
                if t < LH:
                    pending.append(send(wqb, blk_l, ql_send, ql_recv, t, left))
                    pending.append(send(wob, blk_l, ol_send, ol_recv, t, left))
            acc = acc + block_contrib(blk_r)
            if t <= LH:
                acc = acc + block_contrib(blk_l)
        for r in pending:
            r.wait_send()
        o_ref[...] = acc

    out2d = pl.pallas_call(
        body,
        out_shape=jax.ShapeDtypeStruct((B * Sq, Dm), jnp.float32),
        in_specs=[pl.BlockSpec(memory_space=pltpu.VMEM)] * 5,
        out_specs=pl.BlockSpec(memory_space=pltpu.VMEM),
        scratch_shapes=[
            pltpu.VMEM((N_DEV, Dm, H_loc * DH), jnp.bfloat16),
            pltpu.VMEM((N_DEV, H_loc * DH, Dm), jnp.bfloat16),
            pltpu.SemaphoreType.DMA((RH,)),
            pltpu.SemaphoreType.DMA((RH,)),
            pltpu.SemaphoreType.DMA((RH,)),
            pltpu.SemaphoreType.DMA((RH,)),
            pltpu.SemaphoreType.DMA((LH,)),
            pltpu.SemaphoreType.DMA((LH,)),
            pltpu.SemaphoreType.DMA((LH,)),
            pltpu.SemaphoreType.DMA((LH,)),
        ],
        compiler_params=pltpu.CompilerParams(collective_id=0),
    )(x2d, wq_b, kp, vp, wo_b)

    return out2d.reshape(B, Sq, Dm)


# device time: 57790 ns/iter; 2.1888x vs baseline; 1.3054x over previous
import functools

import jax
import jax.numpy as jnp
from jax import lax
from jax.experimental import pallas as pl
from jax.experimental.pallas import tpu as pltpu

N_DEV = 16
NZ = 4
NP = 4
SQ = 128
SKV = 128
DH = 64


def kernel(x, Wq, K_ext, V_ext, Wo):
    B, Sq, Dm = x.shape
    H_loc = Wq.shape[1] // DH
    H = N_DEV * H_loc
    BH = B * H_loc

    my = lax.axis_index("i")

    k_loc = lax.dynamic_slice_in_dim(K_ext, my * B, B, axis=0)
    v_loc = lax.dynamic_slice_in_dim(V_ext, my * B, B, axis=0)
    kp = k_loc.transpose(2, 0, 1, 3).reshape(H * B, SKV, DH).astype(jnp.bfloat16)
    vp = v_loc.transpose(2, 0, 1, 3).reshape(H * B, SKV, DH).astype(jnp.bfloat16)

    x2d = x.reshape(B * Sq, Dm).astype(jnp.bfloat16)
    pk = jnp.concatenate(
        [Wq.astype(jnp.bfloat16), Wo.astype(jnp.bfloat16).T], axis=1
    )
    hw = H_loc * DH

    def body(x_ref, pk_ref, k_ref, v_ref, o_ref, pkb,
             z_s, z_r, r1_s, r1_r, l1_s, l1_r, r2_s, r2_r, l2_s, l2_r):
        me = lax.axis_index("i")
        p = lax.rem(me, NP)
        base = me - p
        pr = base + lax.rem(p + 1, NP)
        pl_ = base + lax.rem(p + 3, NP)
        zr = lax.rem(me + NP, N_DEV)
        zl = lax.rem(me + N_DEV - NP, N_DEV)

        def blk_of(dev, k):
            return lax.rem(dev + N_DEV - NP * k, N_DEV)

        barrier_sem = pltpu.get_barrier_semaphore()
        for nbr in (pr, pl_, zr, zl):
            pl.semaphore_signal(
                barrier_sem, inc=1,
                device_id=(nbr,), device_id_type=pl.DeviceIdType.MESH,
            )
        pl.semaphore_wait(barrier_sem, 4)

        pkb[me] = pk_ref[...]

        xv = x_ref[...]
        qb = lax.broadcasted_iota(jnp.int32, (SQ, SKV), 0) // 64
        kb = lax.broadcasted_iota(jnp.int32, (SQ, SKV), 1) // 64
        mask = (qb == kb) | (kb == 0) | ((qb + kb) % 3 == 0)

        def block_contrib(blk):
            pk_blk = pkb[blk]
            wq_blk = pk_blk[:, :hw]
            wo_blk = pk_blk[:, hw:].T
            q = jnp.dot(xv, wq_blk, preferred_element_type=jnp.float32)
            q = q.astype(jnp.bfloat16)
            q4 = q.reshape(B, Sq, H_loc, DH).transpose(2, 0, 1, 3)
            q4 = q4.reshape(BH, Sq, DH)
            k_blk = k_ref[pl.ds(blk * BH, BH)]
            v_blk = v_ref[pl.ds(blk * BH, BH)]
            scores = lax.dot_general(
                q4, k_blk, (((2,), (2,)), ((0,), (0,))),
                preferred_element_type=jnp.float32,
            ) * 0.125
            scores = jnp.where(mask[None], scores, -1e9)
            m = jnp.max(scores, axis=-1, keepdims=True)
            e = jnp.exp(scores - m)
            w = (e / jnp.sum(e, axis=-1, keepdims=True)).astype(jnp.bfloat16)
            ctx = lax.dot_general(
                w, v_blk, (((2,), (1,)), ((0,), (0,))),
                preferred_element_type=jnp.float32,
            )
            ctx = ctx.astype(jnp.bfloat16).reshape(H_loc, B, Sq, DH)
            ctx2d = ctx.transpose(1, 2, 0, 3).reshape(B * Sq, H_loc * DH)
            return jnp.dot(ctx2d, wo_blk, preferred_element_type=jnp.float32)

        import os as _os
        if _os.environ.get("KERNEL_NO_COMPUTE"):
            block_contrib = lambda blk: jnp.zeros((B * Sq, Dm), jnp.float32)

        pending = []

        def rsend(blk, ss, rs, idx, dev):
            r = pltpu.make_async_remote_copy(
                src_ref=pkb.at[blk], dst_ref=pkb.at[blk],
                send_sem=ss.at[idx], recv_sem=rs.at[idx],
                device_id=(dev,), device_id_type=pl.DeviceIdType.MESH,
            )
            r.start()
            pending.append(r)

        def rwait(blk, ss, rs, idx):
            pltpu.make_async_remote_copy(
                src_ref=pkb.at[blk], dst_ref=pkb.at[blk],
                send_sem=ss.at[idx], recv_sem=rs.at[idx],
                device_id=(pr,), device_id_type=pl.DeviceIdType.MESH,
            ).wait_recv()

        rsend(me, z_s, z_r, 0, zr)
        rsend(me, r1_s, r1_r, 0, pr)
        rsend(me, l1_s, l1_r, 0, pl_)
        acc = block_contrib(me)

        for s in (1, 2, 3):
            bz = blk_of(me, s)
            rwait(bz, z_s, z_r, s - 1)
            if s < NZ - 1:
                rsend(bz, z_s, z_r, s, zr)
            rsend(bz, r1_s, r1_r, s, pr)
            rsend(bz, l1_s, l1_r, s, pl_)
            br = blk_of(pl_, s - 1)
            rwait(br, r1_s, r1_r, s - 1)
            if s - 1 in (1, 2):
                rsend(br, r2_s, r2_r, s - 2, pr)
            bl = blk_of(pr, s - 1)
            rwait(bl, l1_s, l1_r, s - 1)
            if s - 1 == 0:
                rsend(bl, l2_s, l2_r, 0, pl_)
            acc = acc + block_contrib(bz)
            acc = acc + block_contrib(br)
            acc = acc + block_contrib(bl)

        br = blk_of(pl_, 3)
        rwait(br, r1_s, r1_r, 3)
        bl = blk_of(pr, 3)
        rwait(bl, l1_s, l1_r, 3)
        rsend(bl, l2_s, l2_r, 1, pl_)
        acc = acc + block_contrib(br)
        acc = acc + block_contrib(bl)

        ll = base + lax.rem(p + 2, NP)
        b0 = blk_of(ll, 0)
        rwait(b0, l2_s, l2_r, 0)
        acc = acc + block_contrib(b0)
        b1 = blk_of(ll, 1)
        rwait(b1, r2_s, r2_r, 0)
        acc = acc + block_contrib(b1)
        b2 = blk_of(ll, 2)
        rwait(b2, r2_s, r2_r, 1)
        acc = acc + block_contrib(b2)
        b3 = blk_of(ll, 3)
        rwait(b3, l2_s, l2_r, 1)
        acc = acc + block_contrib(b3)

        for r in pending:
            r.wait_send()

        @functools.partial(
            pl.run_scoped, second_barrier=pltpu.SemaphoreType.REGULAR
        )
        def _(second_barrier):
            for nbr in (pr, pl_, zr, zl):
                pl.semaphore_signal(
                    second_barrier, inc=1,
                    device_id=(nbr,), device_id_type=pl.DeviceIdType.MESH,
                )
            pl.semaphore_wait(second_barrier, 4)

        o_ref[...] = acc

    out2d = pl.pallas_call(
        body,
        out_shape=jax.ShapeDtypeStruct((B * Sq, Dm), jnp.float32),
        in_specs=[pl.BlockSpec(memory_space=pltpu.VMEM)] * 4,
        out_specs=pl.BlockSpec(memory_space=pltpu.VMEM),
        scratch_shapes=[
            pltpu.VMEM((N_DEV, Dm, 2 * hw), jnp.bfloat16),
            pltpu.SemaphoreType.DMA((NZ - 1,)),
            pltpu.SemaphoreType.DMA((NZ - 1,)),
            pltpu.SemaphoreType.DMA((NZ,)),
            pltpu.SemaphoreType.DMA((NZ,)),
            pltpu.SemaphoreType.DMA((NZ,)),
            pltpu.SemaphoreType.DMA((NZ,)),
            pltpu.SemaphoreType.DMA((2,)),
            pltpu.SemaphoreType.DMA((2,)),
            pltpu.SemaphoreType.DMA((2,)),
            pltpu.SemaphoreType.DMA((2,)),
        ],
        compiler_params=pltpu.CompilerParams(collective_id=0),
    )(x2d, pk, kp, vp)

    return out2d.reshape(B, Sq, Dm)
